# baseline (speedup 1.0000x reference)
import functools

import jax
import jax.numpy as jnp
from jax import lax
from jax.experimental import pallas as pl
from jax.experimental.pallas import tpu as pltpu
from jax.experimental.pallas import tpu_sc as plsc

N_NODES = 10000
N_EDGES = 320000
D = 128
ED = 16

NC = 2
NS = 16

NPAD = 10240
NHALF = NPAD // NC
NACC = 6144
DUMP = 6000
EPAD = 327680
EW = EPAD // NS
SUB = 16
STEPS = EW // SUB
RT = NACC // NS

_HIGH = jax.lax.Precision.HIGHEST


def _node_pre_body(x_ref, cong_ref, w1s_ref, w1d_ref, w1c_ref, a_ref, b_ref):
    xb = x_ref[...]
    a_ref[...] = (
        jnp.dot(xb, w1s_ref[...], precision=_HIGH, preferred_element_type=jnp.float32)
        + cong_ref[...] * w1c_ref[...]
    )
    b_ref[...] = jnp.dot(
        xb, w1d_ref[...], precision=_HIGH, preferred_element_type=jnp.float32
    )


def _edge_pre_body(attr_ref, w1e_ref, b1_ref, e_ref):
    e_ref[...] = (
        jnp.dot(attr_ref[...], w1e_ref[...], precision=_HIGH,
                preferred_element_type=jnp.float32)
        + b1_ref[...]
    )


def _post_body(s_ref, d_ref, x_ref, w2_ref, b2_ref, u1a_ref, u1b_ref,
               ub1_ref, u2_ref, ub2_ref, o_ref):
    deg = d_ref[...][:, 0:1]
    agg = (
        jnp.dot(s_ref[...], w2_ref[...], precision=_HIGH,
                preferred_element_type=jnp.float32)
        + deg * b2_ref[...]
    )
    h2 = jax.nn.relu(
        jnp.dot(x_ref[...], u1a_ref[...], precision=_HIGH,
                preferred_element_type=jnp.float32)
        + jnp.dot(agg, u1b_ref[...], precision=_HIGH,
                  preferred_element_type=jnp.float32)
        + ub1_ref[...]
    )
    o_ref[...] = (
        jnp.dot(h2, u2_ref[...], precision=_HIGH, preferred_element_type=jnp.float32)
        + ub2_ref[...]
    )


def _sc_edge_body(a_hbm, b_hbm, e_hbm, src_hbm, dst_hbm, ldst_hbm,
                  s_out, d_out,
                  sidx_v, didx_v, lidx_v, a_v, b_v, e_v, ones_v,
                  s_sp, d_sp, sem):
    s = lax.axis_index("s")
    c = lax.axis_index("c")

    zero16 = jnp.zeros((16,), jnp.float32)
    one16 = jnp.ones((16,), jnp.float32)

    def _zrow(r, carry):
        for t in range(D // 16):
            a_v[r, pl.ds(t * 16, 16)] = zero16
        ones_v[r, :] = zero16
        return carry

    lax.fori_loop(0, SUB, _zrow, 0)
    for q in range(RT // SUB):
        off = s * RT + q * SUB
        pltpu.sync_copy(a_v, s_sp.at[pl.ds(off, SUB)])
        pltpu.sync_copy(ones_v, d_sp.at[pl.ds(off, SUB)])

    def _orow(r, carry):
        ones_v[r, :] = one16
        return carry

    lax.fori_loop(0, SUB, _orow, 0)
    plsc.subcore_barrier()

    base = s * EW

    def _outer(n, carry):
        off = base + n * SUB
        pltpu.sync_copy(src_hbm.at[pl.ds(off, SUB)], sidx_v)
        pltpu.sync_copy(dst_hbm.at[pl.ds(off, SUB)], didx_v)
        pltpu.sync_copy(ldst_hbm.at[pl.ds(c * EPAD + off, SUB)], lidx_v)
        pltpu.async_copy(a_hbm.at[sidx_v], a_v, sem).wait()
        pltpu.async_copy(b_hbm.at[didx_v], b_v, sem).wait()
        pltpu.sync_copy(e_hbm.at[pl.ds(off, SUB)], e_v)

        def _crow(r, cc):
            for t in range(D // 16):
                sl = pl.ds(t * 16, 16)
                e_v[r, sl] = jnp.maximum(
                    a_v[r, sl] + b_v[r, sl] + e_v[r, sl], 0.0)
            return cc

        lax.fori_loop(0, SUB, _crow, 0)
        pltpu.sync_copy(e_v, s_sp.at[lidx_v], add=True)
        pltpu.sync_copy(ones_v, d_sp.at[lidx_v], add=True)
        return carry

    lax.fori_loop(0, STEPS, _outer, 0)
    plsc.subcore_barrier()

    off = c * NACC + s * RT
    pltpu.sync_copy(s_sp.at[pl.ds(s * RT, RT)], s_out.at[pl.ds(off, RT)])
    pltpu.sync_copy(d_sp.at[pl.ds(s * RT, RT)], d_out.at[pl.ds(off, RT)])


_sc_edge = functools.partial(
    pl.kernel,
    mesh=plsc.VectorSubcoreMesh(core_axis_name="c", subcore_axis_name="s"),
    out_type=(
        jax.ShapeDtypeStruct((NC * NACC, D), jnp.float32),
        jax.ShapeDtypeStruct((NC * NACC, 16), jnp.float32),
    ),
    scratch_types=[
        pltpu.VMEM((SUB,), jnp.int32),
        pltpu.VMEM((SUB,), jnp.int32),
        pltpu.VMEM((SUB,), jnp.int32),
        pltpu.VMEM((SUB, D), jnp.float32),
        pltpu.VMEM((SUB, D), jnp.float32),
        pltpu.VMEM((SUB, D), jnp.float32),
        pltpu.VMEM((SUB, 16), jnp.float32),
        pltpu.VMEM_SHARED((NACC, D), jnp.float32),
        pltpu.VMEM_SHARED((NACC, 16), jnp.float32),
        pltpu.SemaphoreType.DMA,
    ],
)(_sc_edge_body)


def kernel(x, edge_index, edge_attr, congestion, W1, b1, W2, b2, U1, ub1, U2, ub2):
    x = x.astype(jnp.float32)
    src = edge_index[0].astype(jnp.int32)
    dst = edge_index[1].astype(jnp.int32)

    x_p = jnp.zeros((NPAD, D), jnp.float32).at[:N_NODES].set(x)
    cong_p = jnp.zeros((NPAD, 1), jnp.float32).at[:N_NODES, 0].set(congestion)
    pad = EPAD - N_EDGES
    src_p = jnp.concatenate([src, jnp.zeros((pad,), jnp.int32)])
    dst_p = jnp.concatenate([dst, jnp.full((pad,), N_NODES, jnp.int32)])
    attr_p = jnp.zeros((EPAD, ED), jnp.float32).at[:N_EDGES].set(edge_attr)

    w1s = W1[:D]
    w1d = W1[D:2 * D]
    w1e = W1[2 * D:2 * D + ED]
    w1c = W1[2 * D + ED:]
    u1a = U1[:D]
    u1b = U1[D:]

    full = lambda shape: pl.BlockSpec(shape, lambda i: (0,) * len(shape))

    a_tab, b_tab = pl.pallas_call(
        _node_pre_body,
        grid=(NPAD // 1024,),
        in_specs=[
            pl.BlockSpec((1024, D), lambda i: (i, 0)),
            pl.BlockSpec((1024, 1), lambda i: (i, 0)),
            full((D, D)), full((D, D)), full((1, D)),
        ],
        out_specs=[
            pl.BlockSpec((1024, D), lambda i: (i, 0)),
            pl.BlockSpec((1024, D), lambda i: (i, 0)),
        ],
        out_shape=[
            jax.ShapeDtypeStruct((NPAD, D), jnp.float32),
            jax.ShapeDtypeStruct((NPAD, D), jnp.float32),
        ],
    )(x_p, cong_p, w1s, w1d, w1c)

    e_tab = pl.pallas_call(
        _edge_pre_body,
        grid=(EPAD // 2048,),
        in_specs=[
            pl.BlockSpec((2048, ED), lambda i: (i, 0)),
            full((ED, D)), full((1, D)),
        ],
        out_specs=pl.BlockSpec((2048, D), lambda i: (i, 0)),
        out_shape=jax.ShapeDtypeStruct((EPAD, D), jnp.float32),
    )(attr_p, w1e, b1.reshape(1, D))

    ldst = jnp.concatenate([
        jnp.where(dst_p < NHALF, dst_p, DUMP),
        jnp.where(dst_p >= NHALF, dst_p - NHALF, DUMP),
    ])
    s_flat, d_flat = _sc_edge(a_tab, b_tab, e_tab, src_p, dst_p, ldst)
    s_full = jnp.concatenate(
        [s_flat[:NHALF], s_flat[NACC:NACC + N_NODES - NHALF]])
    d_full = jnp.concatenate(
        [d_flat[:NHALF], d_flat[NACC:NACC + N_NODES - NHALF]])

    out = pl.pallas_call(
        _post_body,
        grid=(N_NODES // 1000,),
        in_specs=[
            pl.BlockSpec((1000, D), lambda i: (i, 0)),
            pl.BlockSpec((1000, 16), lambda i: (i, 0)),
            pl.BlockSpec((1000, D), lambda i: (i, 0)),
            full((D, D)), full((1, D)),
            full((D, D)), full((D, D)), full((1, D)),
            full((D, D)), full((1, D)),
        ],
        out_specs=pl.BlockSpec((1000, D), lambda i: (i, 0)),
        out_shape=jax.ShapeDtypeStruct((N_NODES, D), jnp.float32),
    )(s_full, d_full, x_p[:N_NODES], W2, b2.reshape(1, D),
      u1a, u1b, ub1.reshape(1, D), U2, ub2.reshape(1, D))

    return out


# speedup vs baseline: 1.4426x; 1.4426x over previous
import functools

import jax
import jax.numpy as jnp
from jax import lax
from jax.experimental import pallas as pl
from jax.experimental.pallas import tpu as pltpu
from jax.experimental.pallas import tpu_sc as plsc

N_NODES = 10000
N_EDGES = 320000
D = 128
ED = 16

NC = 2
NS = 16

NPAD = 10240
NHALF = NPAD // NC
NACC = 6144
DUMP = 6000
EPAD = 327680
EW = EPAD // NS
SUB = 32
STEPS = EW // SUB
RT = NACC // NS

_HIGH = jax.lax.Precision.HIGHEST


def _node_pre_body(x_ref, cong_ref, w1s_ref, w1d_ref, w1c_ref, a_ref, b_ref):
    xb = x_ref[...]
    a_ref[...] = (
        jnp.dot(xb, w1s_ref[...], precision=_HIGH, preferred_element_type=jnp.float32)
        + cong_ref[...] * w1c_ref[...]
    )
    b_ref[...] = jnp.dot(
        xb, w1d_ref[...], precision=_HIGH, preferred_element_type=jnp.float32
    )


def _edge_pre_body(attr_ref, w1e_ref, b1_ref, e_ref):
    e_ref[...] = (
        jnp.dot(attr_ref[...], w1e_ref[...], precision=_HIGH,
                preferred_element_type=jnp.float32)
        + b1_ref[...]
    )


def _post_body(s_ref, d_ref, x_ref, w2_ref, b2_ref, u1a_ref, u1b_ref,
               ub1_ref, u2_ref, ub2_ref, o_ref):
    deg = d_ref[...][:, 0:1]
    agg = (
        jnp.dot(s_ref[...], w2_ref[...], precision=_HIGH,
                preferred_element_type=jnp.float32)
        + deg * b2_ref[...]
    )
    h2 = jax.nn.relu(
        jnp.dot(x_ref[...], u1a_ref[...], precision=_HIGH,
                preferred_element_type=jnp.float32)
        + jnp.dot(agg, u1b_ref[...], precision=_HIGH,
                  preferred_element_type=jnp.float32)
        + ub1_ref[...]
    )
    o_ref[...] = (
        jnp.dot(h2, u2_ref[...], precision=_HIGH, preferred_element_type=jnp.float32)
        + ub2_ref[...]
    )


def _sc_edge_body(a_hbm, b_hbm, e_hbm, src_hbm, dst_hbm, ldst_hbm,
                  s_out, d_out,
                  sidx_v, didx_v, lidx_v, a_v, b_v, e_v, ones_v,
                  s_sp, d_sp, sem):
    s = lax.axis_index("s")
    c = lax.axis_index("c")

    zero16 = jnp.zeros((16,), jnp.float32)
    one16 = jnp.ones((16,), jnp.float32)

    def _zrow(r, carry):
        for t in range(D // 16):
            a_v[r, pl.ds(t * 16, 16)] = zero16
        ones_v[r, :] = zero16
        return carry

    lax.fori_loop(0, SUB, _zrow, 0)
    for q in range(RT // SUB):
        off = s * RT + q * SUB
        pltpu.sync_copy(a_v, s_sp.at[pl.ds(off, SUB)])
        pltpu.sync_copy(ones_v, d_sp.at[pl.ds(off, SUB)])

    def _orow(r, carry):
        ones_v[r, :] = one16
        return carry

    lax.fori_loop(0, SUB, _orow, 0)
    plsc.subcore_barrier()

    base = s * EW

    def _outer(n, carry):
        off = base + n * SUB
        pltpu.sync_copy(src_hbm.at[pl.ds(off, SUB)], sidx_v)
        pltpu.sync_copy(dst_hbm.at[pl.ds(off, SUB)], didx_v)
        pltpu.sync_copy(ldst_hbm.at[pl.ds(c * EPAD + off, SUB)], lidx_v)
        pltpu.async_copy(a_hbm.at[sidx_v], a_v, sem).wait()
        pltpu.async_copy(b_hbm.at[didx_v], b_v, sem).wait()
        pltpu.sync_copy(e_hbm.at[pl.ds(off, SUB)], e_v)

        def _crow(r, cc):
            for t in range(D // 16):
                sl = pl.ds(t * 16, 16)
                e_v[r, sl] = jnp.maximum(
                    a_v[r, sl] + b_v[r, sl] + e_v[r, sl], 0.0)
            return cc

        lax.fori_loop(0, SUB, _crow, 0)
        pltpu.sync_copy(e_v, s_sp.at[lidx_v], add=True)
        pltpu.sync_copy(ones_v, d_sp.at[lidx_v], add=True)
        return carry

    lax.fori_loop(0, STEPS, _outer, 0)
    plsc.subcore_barrier()

    off = c * NACC + s * RT
    pltpu.sync_copy(s_sp.at[pl.ds(s * RT, RT)], s_out.at[pl.ds(off, RT)])
    pltpu.sync_copy(d_sp.at[pl.ds(s * RT, RT)], d_out.at[pl.ds(off, RT)])


_sc_edge = functools.partial(
    pl.kernel,
    mesh=plsc.VectorSubcoreMesh(core_axis_name="c", subcore_axis_name="s"),
    out_type=(
        jax.ShapeDtypeStruct((NC * NACC, D), jnp.float32),
        jax.ShapeDtypeStruct((NC * NACC, 16), jnp.float32),
    ),
    scratch_types=[
        pltpu.VMEM((SUB,), jnp.int32),
        pltpu.VMEM((SUB,), jnp.int32),
        pltpu.VMEM((SUB,), jnp.int32),
        pltpu.VMEM((SUB, D), jnp.float32),
        pltpu.VMEM((SUB, D), jnp.float32),
        pltpu.VMEM((SUB, D), jnp.float32),
        pltpu.VMEM((SUB, 16), jnp.float32),
        pltpu.VMEM_SHARED((NACC, D), jnp.float32),
        pltpu.VMEM_SHARED((NACC, 16), jnp.float32),
        pltpu.SemaphoreType.DMA,
    ],
)(_sc_edge_body)


def kernel(x, edge_index, edge_attr, congestion, W1, b1, W2, b2, U1, ub1, U2, ub2):
    x = x.astype(jnp.float32)
    src = edge_index[0].astype(jnp.int32)
    dst = edge_index[1].astype(jnp.int32)

    x_p = jnp.zeros((NPAD, D), jnp.float32).at[:N_NODES].set(x)
    cong_p = jnp.zeros((NPAD, 1), jnp.float32).at[:N_NODES, 0].set(congestion)
    pad = EPAD - N_EDGES
    src_p = jnp.concatenate([src, jnp.zeros((pad,), jnp.int32)])
    dst_p = jnp.concatenate([dst, jnp.full((pad,), N_NODES, jnp.int32)])
    attr_p = jnp.zeros((EPAD, ED), jnp.float32).at[:N_EDGES].set(edge_attr)

    w1s = W1[:D]
    w1d = W1[D:2 * D]
    w1e = W1[2 * D:2 * D + ED]
    w1c = W1[2 * D + ED:]
    u1a = U1[:D]
    u1b = U1[D:]

    full = lambda shape: pl.BlockSpec(shape, lambda i: (0,) * len(shape))

    a_tab, b_tab = pl.pallas_call(
        _node_pre_body,
        grid=(NPAD // 1024,),
        in_specs=[
            pl.BlockSpec((1024, D), lambda i: (i, 0)),
            pl.BlockSpec((1024, 1), lambda i: (i, 0)),
            full((D, D)), full((D, D)), full((1, D)),
        ],
        out_specs=[
            pl.BlockSpec((1024, D), lambda i: (i, 0)),
            pl.BlockSpec((1024, D), lambda i: (i, 0)),
        ],
        out_shape=[
            jax.ShapeDtypeStruct((NPAD, D), jnp.float32),
            jax.ShapeDtypeStruct((NPAD, D), jnp.float32),
        ],
    )(x_p, cong_p, w1s, w1d, w1c)

    e_tab = pl.pallas_call(
        _edge_pre_body,
        grid=(EPAD // 2048,),
        in_specs=[
            pl.BlockSpec((2048, ED), lambda i: (i, 0)),
            full((ED, D)), full((1, D)),
        ],
        out_specs=pl.BlockSpec((2048, D), lambda i: (i, 0)),
        out_shape=jax.ShapeDtypeStruct((EPAD, D), jnp.float32),
    )(attr_p, w1e, b1.reshape(1, D))

    ldst = jnp.concatenate([
        jnp.where(dst_p < NHALF, dst_p, DUMP),
        jnp.where(dst_p >= NHALF, dst_p - NHALF, DUMP),
    ])
    s_flat, d_flat = _sc_edge(a_tab, b_tab, e_tab, src_p, dst_p, ldst)
    s_full = jnp.concatenate(
        [s_flat[:NHALF], s_flat[NACC:NACC + N_NODES - NHALF]])
    d_full = jnp.concatenate(
        [d_flat[:NHALF], d_flat[NACC:NACC + N_NODES - NHALF]])

    out = pl.pallas_call(
        _post_body,
        grid=(N_NODES // 1000,),
        in_specs=[
            pl.BlockSpec((1000, D), lambda i: (i, 0)),
            pl.BlockSpec((1000, 16), lambda i: (i, 0)),
            pl.BlockSpec((1000, D), lambda i: (i, 0)),
            full((D, D)), full((1, D)),
            full((D, D)), full((D, D)), full((1, D)),
            full((D, D)), full((1, D)),
        ],
        out_specs=pl.BlockSpec((1000, D), lambda i: (i, 0)),
        out_shape=jax.ShapeDtypeStruct((N_NODES, D), jnp.float32),
    )(s_full, d_full, x_p[:N_NODES], W2, b2.reshape(1, D),
      u1a, u1b, ub1.reshape(1, D), U2, ub2.reshape(1, D))

    return out
